# Initial kernel scaffold; baseline (speedup 1.0000x reference)
import jax
import jax.numpy as jnp
from jax.experimental import pallas as pl


def kernel(x, edge_index, lin_w, film_w, film_b, skip_w, film_skip_w, bn_w, bn_b):
    raise NotImplementedError("write your pallas kernel here")


# speedup vs baseline: 3.8756x; 3.8756x over previous
import functools

import jax
import jax.numpy as jnp
from jax import lax
from jax.experimental import pallas as pl
from jax.experimental.pallas import tpu as pltpu
from jax.experimental.pallas import tpu_sc as plsc

_N = 10000
_E = 320000
_D = 128
_EPS = 1e-5

_NC = 2
_NS = 16
_TILES = _NC * _NS
_EPT = _E // _TILES
_C = 80
_CHUNKS = _EPT // _C
_RCH = 80
_NRCH = _N // _RCH
_QMAX = (_NRCH + _NS - 1) // _NS
_BN = 400


def _dense_body(act, stats_ref, bnw_ref, bnb_ref, h_ref, w_ref, fb_ref,
                hl_ref, gb_ref, out_ref):
    m = stats_ref[0:1, :] * (1.0 / _N)
    ms = stats_ref[1:2, :] * (1.0 / _N)
    v = ms - m * m
    s = bnw_ref[...] * lax.rsqrt(v + _EPS)
    t = bnb_ref[...] - m * s
    hn = h_ref[...] * s + t
    z = jnp.dot(hn, w_ref[...], preferred_element_type=jnp.float32)
    hl_ref[...] = z[:, 0:_D]
    gb_ref[...] = z[:, 2 * _D:4 * _D] + fb_ref[...]
    o = z[:, 5 * _D:6 * _D] * z[:, _D:2 * _D] + z[:, 4 * _D:5 * _D]
    out_ref[...] = jnp.maximum(o, 0.0) if act else o


def _dense_call(act, stats, bnw, bnb, h, wcat, fb):
    grid = _N // _BN
    return pl.pallas_call(
        functools.partial(_dense_body, act),
        grid=(grid,),
        in_specs=[
            pl.BlockSpec((2, _D), lambda i: (0, 0)),
            pl.BlockSpec((1, _D), lambda i: (0, 0)),
            pl.BlockSpec((1, _D), lambda i: (0, 0)),
            pl.BlockSpec((_BN, _D), lambda i: (i, 0)),
            pl.BlockSpec((_D, 6 * _D), lambda i: (0, 0)),
            pl.BlockSpec((1, 2 * _D), lambda i: (0, 0)),
        ],
        out_specs=[
            pl.BlockSpec((_BN, _D), lambda i: (i, 0)),
            pl.BlockSpec((_BN, 2 * _D), lambda i: (i, 0)),
            pl.BlockSpec((_BN, _D), lambda i: (i, 0)),
        ],
        out_shape=[
            jax.ShapeDtypeStruct((_N, _D), jnp.float32),
            jax.ShapeDtypeStruct((_N, 2 * _D), jnp.float32),
            jax.ShapeDtypeStruct((_N, _D), jnp.float32),
        ],
    )(stats, bnw, bnb, h, wcat, fb)


def _comb_body(out_ref, acc_ref, deg_ref, y_ref, stats_ref, sums):
    i = pl.program_id(0)
    a = acc_ref[0] + acc_ref[1]
    dg = deg_ref[0, :, 0:1] + deg_ref[1, :, 0:1]
    y = out_ref[...] + a / jnp.maximum(dg, 1.0)
    y_ref[...] = y

    @pl.when(i == 0)
    def _():
        sums[...] = jnp.zeros_like(sums)

    sums[0:1, :] += jnp.sum(y, axis=0, keepdims=True)
    sums[1:2, :] += jnp.sum(y * y, axis=0, keepdims=True)

    @pl.when(i == pl.num_programs(0) - 1)
    def _():
        stats_ref[...] = sums[0:2, :]


def _comb_call(out, acc, deg):
    grid = _N // _BN
    return pl.pallas_call(
        _comb_body,
        grid=(grid,),
        in_specs=[
            pl.BlockSpec((_BN, _D), lambda i: (i, 0)),
            pl.BlockSpec((2, _BN, _D), lambda i: (0, i, 0)),
            pl.BlockSpec((2, _BN, _D), lambda i: (0, i, 0)),
        ],
        out_specs=[
            pl.BlockSpec((_BN, _D), lambda i: (i, 0)),
            pl.BlockSpec((2, _D), lambda i: (0, 0)),
        ],
        out_shape=[
            jax.ShapeDtypeStruct((_N, _D), jnp.float32),
            jax.ShapeDtypeStruct((2, _D), jnp.float32),
        ],
        scratch_shapes=[pltpu.VMEM((8, _D), jnp.float32)],
    )(out, acc, deg)


def _final_body(out_ref, gb_ref, acc_ref, deg_ref, y_ref):
    a = acc_ref[0] + acc_ref[1]
    dg = deg_ref[0, :, 0:1] + deg_ref[1, :, 0:1]
    beta = gb_ref[:, 0:_D]
    gamma = gb_ref[:, _D:2 * _D]
    has = (dg > 0.0).astype(jnp.float32)
    y_ref[...] = out_ref[...] + gamma * (a / jnp.maximum(dg, 1.0)) + beta * has


def _final_call(out, gb, acc, deg):
    grid = _N // _BN
    return pl.pallas_call(
        _final_body,
        grid=(grid,),
        in_specs=[
            pl.BlockSpec((_BN, _D), lambda i: (i, 0)),
            pl.BlockSpec((_BN, 2 * _D), lambda i: (i, 0)),
            pl.BlockSpec((2, _BN, _D), lambda i: (0, i, 0)),
            pl.BlockSpec((2, _BN, _D), lambda i: (0, i, 0)),
        ],
        out_specs=pl.BlockSpec((_BN, _D), lambda i: (i, 0)),
        out_shape=jax.ShapeDtypeStruct((_N, _D), jnp.float32),
    )(out, gb, acc, deg)


def _sc_mesh():
    return plsc.VectorSubcoreMesh(core_axis_name="c", subcore_axis_name="s")


def _zero_rows(ref, nrows):
    def zrow(r, carry):
        for j in range(ref.shape[1] // 16):
            ref[r, pl.ds(j * 16, 16)] = jnp.zeros((16,), jnp.float32)
        return carry
    lax.fori_loop(0, nrows, zrow, None)


def _init_shared(acc_sh, stage, s):
    _zero_rows(stage, _RCH)

    def cp(q, carry):
        cid = q * _NS + s

        @pl.when(cid < _NRCH)
        def _():
            pltpu.sync_copy(
                stage, acc_sh.at[pl.ds(pl.multiple_of(cid * _RCH, 8), _RCH)])
        return carry

    lax.fori_loop(0, _QMAX, cp, None)


def _copy_out(acc_sh, stage, out_hbm, c, s):
    def cp(q, carry):
        cid = q * _NS + s

        @pl.when(cid < _NRCH)
        def _():
            r0 = pl.multiple_of(cid * _RCH, 8)
            pltpu.sync_copy(acc_sh.at[pl.ds(r0, _RCH)], stage)
            pltpu.sync_copy(stage, out_hbm.at[c, pl.ds(r0, _RCH)])
        return carry

    lax.fori_loop(0, _QMAX, cp, None)


def _sc_msg(hl, gb, src, dst):
    @functools.partial(
        pl.kernel,
        out_type=jax.ShapeDtypeStruct((_NC, _N, _D), jnp.float32),
        mesh=_sc_mesh(),
        scratch_types=[
            pltpu.VMEM((_C,), jnp.int32),
            pltpu.VMEM((_C,), jnp.int32),
            pltpu.VMEM((_C, _D), jnp.float32),
            pltpu.VMEM((_C, 2 * _D), jnp.float32),
            pltpu.VMEM((_RCH, _D), jnp.float32),
            pltpu.VMEM_SHARED((_N, _D), jnp.float32),
            pltpu.SemaphoreType.DMA,
            pltpu.SemaphoreType.DMA,
        ],
    )
    def k(hl_hbm, gb_hbm, src_hbm, dst_hbm, acc_hbm,
          src_idx, dst_idx, hrow, gbrow, stage, acc_sh, sem1, sem2):
        c = lax.axis_index("c")
        s = lax.axis_index("s")
        tid = c * _NS + s
        _init_shared(acc_sh, stage, s)
        plsc.subcore_barrier()

        ebase = tid * _EPT

        def chunk(kk, carry):
            base = pl.multiple_of(ebase + kk * _C, 8)
            pltpu.sync_copy(src_hbm.at[pl.ds(base, _C)], src_idx)
            pltpu.sync_copy(dst_hbm.at[pl.ds(base, _C)], dst_idx)
            cp1 = pltpu.async_copy(hl_hbm.at[src_idx], hrow, sem1)
            cp2 = pltpu.async_copy(gb_hbm.at[dst_idx], gbrow, sem2)
            cp1.wait()
            cp2.wait()

            def row(r, carry2):
                for j in range(_D // 16):
                    h = hrow[r, pl.ds(j * 16, 16)]
                    b = gbrow[r, pl.ds(j * 16, 16)]
                    g = gbrow[r, pl.ds(_D + j * 16, 16)]
                    hrow[r, pl.ds(j * 16, 16)] = jnp.maximum(g * h + b, 0.0)
                return carry2

            lax.fori_loop(0, _C, row, None)
            pltpu.sync_copy(hrow, acc_sh.at[dst_idx], add=True)
            return carry

        lax.fori_loop(0, _CHUNKS, chunk, None)
        plsc.subcore_barrier()
        _copy_out(acc_sh, stage, acc_hbm, c, s)

    return k(hl, gb, src, dst)


def _sc_sum(hl, src, dst):
    @functools.partial(
        pl.kernel,
        out_type=jax.ShapeDtypeStruct((_NC, _N, _D), jnp.float32),
        mesh=_sc_mesh(),
        scratch_types=[
            pltpu.VMEM((_C,), jnp.int32),
            pltpu.VMEM((_C,), jnp.int32),
            pltpu.VMEM((_C, _D), jnp.float32),
            pltpu.VMEM((_RCH, _D), jnp.float32),
            pltpu.VMEM_SHARED((_N, _D), jnp.float32),
            pltpu.SemaphoreType.DMA,
        ],
    )
    def k(hl_hbm, src_hbm, dst_hbm, acc_hbm,
          src_idx, dst_idx, hrow, stage, acc_sh, sem1):
        c = lax.axis_index("c")
        s = lax.axis_index("s")
        tid = c * _NS + s
        _init_shared(acc_sh, stage, s)
        plsc.subcore_barrier()

        ebase = tid * _EPT

        def chunk(kk, carry):
            base = pl.multiple_of(ebase + kk * _C, 8)
            pltpu.sync_copy(src_hbm.at[pl.ds(base, _C)], src_idx)
            pltpu.sync_copy(dst_hbm.at[pl.ds(base, _C)], dst_idx)
            pltpu.async_copy(hl_hbm.at[src_idx], hrow, sem1).wait()
            pltpu.sync_copy(hrow, acc_sh.at[dst_idx], add=True)
            return carry

        lax.fori_loop(0, _CHUNKS, chunk, None)
        plsc.subcore_barrier()
        _copy_out(acc_sh, stage, acc_hbm, c, s)

    return k(hl, src, dst)


def _sc_deg(dst):
    @functools.partial(
        pl.kernel,
        out_type=jax.ShapeDtypeStruct((_NC, _N, _D), jnp.float32),
        mesh=_sc_mesh(),
        scratch_types=[
            pltpu.VMEM((_C,), jnp.int32),
            pltpu.VMEM((_C, _D), jnp.float32),
            pltpu.VMEM((_RCH, _D), jnp.float32),
            pltpu.VMEM_SHARED((_N, _D), jnp.float32),
        ],
    )
    def k(dst_hbm, deg_hbm, dst_idx, ones_b, stage, deg_sh):
        c = lax.axis_index("c")
        s = lax.axis_index("s")
        tid = c * _NS + s
        _init_shared(deg_sh, stage, s)

        def onesr(r, carry):
            for j in range(_D // 16):
                ones_b[r, pl.ds(j * 16, 16)] = jnp.ones((16,), jnp.float32)
            return carry

        lax.fori_loop(0, _C, onesr, None)
        plsc.subcore_barrier()

        ebase = tid * _EPT

        def chunk(kk, carry):
            base = pl.multiple_of(ebase + kk * _C, 8)
            pltpu.sync_copy(dst_hbm.at[pl.ds(base, _C)], dst_idx)
            pltpu.sync_copy(ones_b, deg_sh.at[dst_idx], add=True)
            return carry

        lax.fori_loop(0, _CHUNKS, chunk, None)
        plsc.subcore_barrier()
        _copy_out(deg_sh, stage, deg_hbm, c, s)

    return k(dst)


def kernel(x, edge_index, lin_w, film_w, film_b, skip_w, film_skip_w, bn_w, bn_b):
    f32 = jnp.float32
    src = edge_index[0]
    dst = edge_index[1]
    wcats = [
        jnp.concatenate(
            [lin_w[i].T, skip_w[i].T, film_w[i].T, film_skip_w[i].T], axis=1)
        for i in range(3)
    ]
    fbs = [film_b[i].reshape(1, 2 * _D) for i in range(3)]

    deg = _sc_deg(dst)

    stats = jnp.concatenate(
        [jnp.zeros((1, _D), f32), jnp.full((1, _D), _N * (1.0 - _EPS), f32)],
        axis=0)
    bnw_id = jnp.ones((1, _D), f32)
    bnb_id = jnp.zeros((1, _D), f32)

    hl, gb, out = _dense_call(True, stats, bnw_id, bnb_id, x, wcats[0], fbs[0])
    acc = _sc_msg(hl, gb, src, dst)
    h, stats = _comb_call(out, acc, deg)

    hl, gb, out = _dense_call(True, stats, bn_w[0:1], bn_b[0:1], h, wcats[1], fbs[1])
    acc = _sc_msg(hl, gb, src, dst)
    h, stats = _comb_call(out, acc, deg)

    hl, gb, out = _dense_call(False, stats, bn_w[1:2], bn_b[1:2], h, wcats[2], fbs[2])
    acc = _sc_sum(hl, src, dst)
    return _final_call(out, gb, acc, deg)
